# baseline (speedup 1.0000x reference)
import functools

import jax
import jax.numpy as jnp
from jax import lax
from jax.experimental import pallas as pl
from jax.experimental.pallas import tpu as pltpu
from jax.experimental.pallas import tpu_sc as plsc

N_NODES = 10000
N_EDGES = 160000
D_IN = 256
D_EMB = 512
N_GRAPHS = 128

W = 128
NSUB = 16
NCORE = 2
E_PER_SUB = 10240
N_WIN = E_PER_SUB // 128
E_PAD = E_PER_SUB * NSUB
ACC_ROWS = 10112
ZT = 79
ZPS = 5
OT = 78
OPS = 5


def _make_sc_agg(n_chunks, with_deg):
    mesh = plsc.VectorSubcoreMesh(core_axis_name="c", subcore_axis_name="s")

    out_type = [jax.ShapeDtypeStruct((n_chunks * N_NODES, W), jnp.float32)]
    if with_deg:
        out_type.append(jax.ShapeDtypeStruct((N_NODES, W), jnp.float32))

    scratch = [
        pltpu.VMEM((N_WIN, 128), jnp.int32),
        pltpu.VMEM((2, 128), jnp.int32),
        pltpu.VMEM((2, 128), jnp.int32),
        pltpu.VMEM((2, 128, W), jnp.float32),
        pltpu.SemaphoreType.DMA,
        pltpu.SemaphoreType.DMA,
        pltpu.VMEM_SHARED((ACC_ROWS, W), jnp.float32),
    ]

    def body(*refs):
        n_out = 2 if with_deg else 1
        flat_in, pk2d = refs[0], refs[1]
        outs = refs[2:2 + n_out]
        flat_out = outs[0]
        deg_out = outs[1] if with_deg else None
        pk_v, sidx, didx, rows, sem_g, sem_s, acc = refs[2 + n_out:]

        c = lax.axis_index("c")
        s = lax.axis_index("s")
        cc = n_chunks // NCORE

        z16 = jnp.zeros((16,), jnp.float32)
        o16 = jnp.ones((16,), jnp.float32)

        pltpu.sync_copy(pk2d.at[pl.ds(s * N_WIN, N_WIN)], pk_v)

        def fill_rows(v16):
            def f_body(t, carry):
                rows[0, t // 8, pl.ds((t % 8) * 16, 16)] = v16
                return carry

            lax.fori_loop(0, 128 * 8, f_body, 0)

        def run_chunk(k, gather, out_hbm, out_base):
            koff = jnp.full((16,), k * N_NODES, jnp.int32)

            def unpack(w, p):
                def u_body(t, carry):
                    cs = pl.ds(t * 16, 16)
                    pk = pk_v[w, cs]
                    didx[p, cs] = pk >> 16
                    if gather:
                        sidx[p, cs] = (pk & 0xFFFF) + koff
                    return carry

                lax.fori_loop(0, 8, u_body, 0)

            fill_rows(z16)

            def z_copy(t, carry):
                pltpu.sync_copy(rows.at[0], acc.at[pl.ds(t * 128, 128)])
                return carry

            zlo = s * ZPS
            lax.fori_loop(zlo, jnp.minimum(ZT, zlo + ZPS), z_copy, 0)
            plsc.subcore_barrier()

            unpack(0, 0)
            if gather:
                pltpu.async_copy(flat_in.at[sidx.at[0]], rows.at[0],
                                 sem_g).wait()
            else:
                fill_rows(o16)

            def w_body(w, carry):
                p = lax.rem(w, 2)
                q = lax.rem(w + 1, 2)
                rbuf = 0 if not gather else p
                sc = pltpu.async_copy(rows.at[rbuf], acc.at[didx.at[p]],
                                      sem_s, add=True)

                @pl.when(w < N_WIN - 1)
                def _():
                    unpack(w + 1, q)
                    if gather:
                        pltpu.async_copy(flat_in.at[sidx.at[q]],
                                         rows.at[q], sem_g).wait()

                sc.wait()
                return carry

            lax.fori_loop(0, N_WIN, w_body, 0)
            plsc.subcore_barrier()

            def o_copy(t, carry):
                b = t * 128
                pltpu.sync_copy(acc.at[pl.ds(b, 128)], rows.at[0])
                pltpu.sync_copy(rows.at[0],
                                out_hbm.at[pl.ds(out_base + b, 128)])
                return carry

            lo = s * OPS
            lax.fori_loop(lo, jnp.minimum(OT, lo + OPS), o_copy, 0)

            @pl.when(s == NSUB - 1)
            def _():
                tb = OT * 128
                pltpu.sync_copy(acc.at[pl.ds(tb, 16)],
                                rows.at[0, pl.ds(0, 16)])
                pltpu.sync_copy(rows.at[0, pl.ds(0, 16)],
                                out_hbm.at[pl.ds(out_base + tb, 16)])
            plsc.subcore_barrier()

        for j in range(cc):
            k = c * cc + j
            run_chunk(k, True, flat_out, k * N_NODES)

        if with_deg:
            @pl.when(c == 0)
            def _():
                run_chunk(0, False, deg_out, 0)

    return pl.kernel(body, out_type=tuple(out_type), mesh=mesh,
                     scratch_types=scratch,
                     name=f"sc_agg{n_chunks}" + ("_deg" if with_deg else ""))


def _tc_r(nc, h3, w_r, b_row):
    r_blk = 400
    n_out = D_EMB // W

    def body(h_ref, w_ref, b_ref, out_ref):
        hcat = jnp.concatenate([h_ref[j] for j in range(nc)], axis=1)
        o = jnp.dot(hcat, w_ref[...], preferred_element_type=jnp.float32)
        o = o + b_ref[...]
        for j in range(n_out):
            out_ref[j] = o[:, j * W:(j + 1) * W]

    in_specs = [pl.BlockSpec((nc, r_blk, W), lambda i: (0, i, 0)),
                pl.BlockSpec(w_r.shape, lambda i: (0, 0)),
                pl.BlockSpec((1, D_EMB), lambda i: (0, 0))]
    return pl.pallas_call(
        body, grid=(N_NODES // r_blk,), in_specs=in_specs,
        out_specs=pl.BlockSpec((n_out, r_blk, W), lambda i: (0, i, 0)),
        out_shape=jax.ShapeDtypeStruct((n_out, N_NODES, W), jnp.float32),
    )(h3, w_r, b_row)


def _tc_l(nc, agg3, deg, r3, w_l):
    r_blk = 400
    n_out = D_EMB // W

    def body(a_ref, d_ref, r_ref, w_ref, out_ref):
        rdeg = 1.0 / jnp.maximum(d_ref[:, 0:1], 1.0)
        mean = jnp.concatenate([a_ref[j] for j in range(nc)], axis=1) * rdeg
        o = jnp.dot(mean, w_ref[...], preferred_element_type=jnp.float32)
        for j in range(n_out):
            out_ref[j] = o[:, j * W:(j + 1) * W] + r_ref[j]

    n_r = D_EMB // W
    in_specs = [pl.BlockSpec((nc, r_blk, W), lambda i: (0, i, 0)),
                pl.BlockSpec((r_blk, W), lambda i: (i, 0)),
                pl.BlockSpec((n_r, r_blk, W), lambda i: (0, i, 0)),
                pl.BlockSpec(w_l.shape, lambda i: (0, 0))]
    return pl.pallas_call(
        body, grid=(N_NODES // r_blk,), in_specs=in_specs,
        out_specs=pl.BlockSpec((n_out, r_blk, W), lambda i: (0, i, 0)),
        out_shape=jax.ShapeDtypeStruct((n_out, N_NODES, W), jnp.float32),
    )(agg3, deg, r3, w_l)


def _tc_pool_mlp(h0, h1, h2, batch3, fc1_w, fc1_b, fc2_w, fc2_b):
    r_blk = 80
    n_blk = N_NODES // r_blk
    d_cat = 3 * D_EMB
    nc = D_EMB // W

    def body(h0_r, h1_r, h2_r, b_ref, w1, b1, w2, b2, out, acc):
        i = pl.program_id(0)

        @pl.when(i == 0)
        def _():
            acc[...] = jnp.full((N_GRAPHS, d_cat), -jnp.inf, jnp.float32)

        rows = jnp.concatenate(
            [r[j] for r in (h0_r, h1_r, h2_r) for j in range(nc)], axis=1)
        gids = b_ref[0]
        g_lo = b_ref[0, 0, 0]
        g_hi = b_ref[0, r_blk - 1, 0]

        def g_body(g, carry):
            m = gids == g
            colmax = jnp.max(jnp.where(m, rows, -jnp.inf), axis=0,
                             keepdims=True)
            cur = acc[pl.ds(g, 1), :]
            acc[pl.ds(g, 1), :] = jnp.maximum(cur, colmax)
            return carry

        lax.fori_loop(g_lo, g_hi + 1, g_body, 0)

        @pl.when(i == n_blk - 1)
        def _():
            p = acc[...]
            z = jnp.maximum(
                jnp.dot(p, w1[...], preferred_element_type=jnp.float32)
                + b1[...], 0.0)
            out[...] = (jnp.dot(z, w2[...], preferred_element_type=jnp.float32)
                        + b2[...])

    h_spec = pl.BlockSpec((nc, r_blk, W), lambda i: (0, i, 0))
    in_specs = [h_spec, h_spec, h_spec,
                pl.BlockSpec((1, r_blk, 1), lambda i: (i, 0, 0)),
                pl.BlockSpec(fc1_w.shape, lambda i: (0, 0)),
                pl.BlockSpec((1, D_EMB), lambda i: (0, 0)),
                pl.BlockSpec(fc2_w.shape, lambda i: (0, 0)),
                pl.BlockSpec((1, 10), lambda i: (0, 0))]
    return pl.pallas_call(
        body, grid=(n_blk,),
        in_specs=in_specs,
        out_specs=pl.BlockSpec((N_GRAPHS, 10), lambda i: (0, 0)),
        out_shape=jax.ShapeDtypeStruct((N_GRAPHS, 10), jnp.float32),
        scratch_shapes=[pltpu.VMEM((N_GRAPHS, d_cat), jnp.float32)],
    )(h0, h1, h2, batch3, fc1_w, fc1_b, fc2_w, fc2_b)


def kernel(x, edge_index, batch, W_l0, b_l0, W_r0, W_l1, b_l1, W_r1,
           W_l2, b_l2, W_r2, fc1_W, fc1_b, fc2_W, fc2_b):
    src, dst = edge_index[0], edge_index[1]
    npad = E_PAD - N_EDGES
    pad_ar = jnp.arange(npad, dtype=jnp.int32)
    src_p = jnp.concatenate([src, (pad_ar * 37) % N_NODES])
    dst_p = jnp.concatenate([dst, N_NODES + (pad_ar % 16)])
    pk2d = ((dst_p << 16) | src_p).reshape(E_PAD // 128, 128)

    nc_in = D_IN // W
    nc_emb = D_EMB // W

    x3 = x.reshape(N_NODES, nc_in, W).transpose(1, 0, 2)
    x_flat = x3.reshape(nc_in * N_NODES, W)

    sc_in = _make_sc_agg(nc_in, with_deg=True)
    sc_emb = _make_sc_agg(nc_emb, with_deg=False)

    agg0_flat, deg = sc_in(x_flat, pk2d)
    r0 = _tc_r(nc_in, x3, W_r0, b_l0.reshape(1, D_EMB))
    h0 = _tc_l(nc_in, agg0_flat.reshape(nc_in, N_NODES, W), deg, r0, W_l0)

    agg1 = sc_emb(h0.reshape(nc_emb * N_NODES, W), pk2d)[0]
    r1 = _tc_r(nc_emb, h0, W_r1, b_l1.reshape(1, D_EMB))
    h1 = _tc_l(nc_emb, agg1.reshape(nc_emb, N_NODES, W), deg, r1, W_l1)

    agg2 = sc_emb(h1.reshape(nc_emb * N_NODES, W), pk2d)[0]
    r2 = _tc_r(nc_emb, h1, W_r2, b_l2.reshape(1, D_EMB))
    h2 = _tc_l(nc_emb, agg2.reshape(nc_emb, N_NODES, W), deg, r2, W_l2)

    batch3 = batch.reshape(N_NODES // 80, 80, 1)
    return _tc_pool_mlp(h0, h1, h2, batch3,
                        fc1_W, fc1_b.reshape(1, D_EMB),
                        fc2_W, fc2_b.reshape(1, 10))


# speedup vs baseline: 1.0014x; 1.0014x over previous
import functools

import jax
import jax.numpy as jnp
from jax import lax
from jax.experimental import pallas as pl
from jax.experimental.pallas import tpu as pltpu
from jax.experimental.pallas import tpu_sc as plsc

N_NODES = 10000
N_EDGES = 160000
D_IN = 256
D_EMB = 512
N_GRAPHS = 128

W = 128
NSUB = 16
NCORE = 2
E_PER_SUB = 10240
N_WIN = E_PER_SUB // 128
E_PAD = E_PER_SUB * NSUB
ACC_ROWS = 10112
ZT = 79
ZPS = 5
OT = 78
OPS = 5


def _make_sc_agg(n_chunks, with_deg):
    mesh = plsc.VectorSubcoreMesh(core_axis_name="c", subcore_axis_name="s")

    out_type = [jax.ShapeDtypeStruct((n_chunks * N_NODES, W), jnp.float32)]
    if with_deg:
        out_type.append(jax.ShapeDtypeStruct((N_NODES, W), jnp.float32))

    scratch = [
        pltpu.VMEM((N_WIN, 128), jnp.int32),
        pltpu.VMEM((2, 128), jnp.int32),
        pltpu.VMEM((2, 128), jnp.int32),
        pltpu.VMEM((2, 128, W), jnp.float32),
        pltpu.SemaphoreType.DMA,
        pltpu.SemaphoreType.DMA,
        pltpu.VMEM_SHARED((ACC_ROWS, W), jnp.float32),
    ]

    def body(*refs):
        n_out = 2 if with_deg else 1
        flat_in, pk2d = refs[0], refs[1]
        outs = refs[2:2 + n_out]
        flat_out = outs[0]
        deg_out = outs[1] if with_deg else None
        pk_v, sidx, didx, rows, sem_g, sem_s, acc = refs[2 + n_out:]

        c = lax.axis_index("c")
        s = lax.axis_index("s")
        cc = n_chunks // NCORE

        z16 = jnp.zeros((16,), jnp.float32)
        o16 = jnp.ones((16,), jnp.float32)

        pltpu.sync_copy(pk2d.at[pl.ds(s * N_WIN, N_WIN)], pk_v)

        def fill_rows(v16):
            def f_body(t, carry):
                rows[0, t // 8, pl.ds((t % 8) * 16, 16)] = v16
                return carry

            lax.fori_loop(0, 128 * 8, f_body, 0)

        def run_chunk(k, gather, out_hbm, out_base):
            koff = jnp.full((16,), k * N_NODES, jnp.int32)

            def unpack(w, p):
                def u_body(t, carry):
                    cs = pl.ds(t * 16, 16)
                    pk = pk_v[w, cs]
                    didx[p, cs] = pk >> 16
                    if gather:
                        sidx[p, cs] = (pk & 0xFFFF) + koff
                    return carry

                lax.fori_loop(0, 8, u_body, 0)

            fill_rows(z16)

            def z_copy(t, carry):
                pltpu.sync_copy(rows.at[0], acc.at[pl.ds(t * 128, 128)])
                return carry

            zlo = s * ZPS
            lax.fori_loop(zlo, jnp.minimum(ZT, zlo + ZPS), z_copy, 0)
            plsc.subcore_barrier()

            unpack(0, 0)
            if gather:
                pltpu.async_copy(flat_in.at[sidx.at[0]], rows.at[0],
                                 sem_g).wait()
            else:
                fill_rows(o16)

            def w_body(w, carry):
                p = lax.rem(w, 2)
                q = lax.rem(w + 1, 2)
                rbuf = 0 if not gather else p
                sc = pltpu.async_copy(rows.at[rbuf], acc.at[didx.at[p]],
                                      sem_s, add=True)

                @pl.when(w < N_WIN - 1)
                def _():
                    unpack(w + 1, q)
                    if gather:
                        pltpu.async_copy(flat_in.at[sidx.at[q]],
                                         rows.at[q], sem_g).wait()

                sc.wait()
                return carry

            lax.fori_loop(0, N_WIN, w_body, 0)
            plsc.subcore_barrier()

            def o_copy(t, carry):
                b = t * 128
                pltpu.sync_copy(acc.at[pl.ds(b, 128)], rows.at[0])
                pltpu.sync_copy(rows.at[0],
                                out_hbm.at[pl.ds(out_base + b, 128)])
                return carry

            lo = s * OPS
            lax.fori_loop(lo, jnp.minimum(OT, lo + OPS), o_copy, 0)

            @pl.when(s == NSUB - 1)
            def _():
                tb = OT * 128
                pltpu.sync_copy(acc.at[pl.ds(tb, 16)],
                                rows.at[0, pl.ds(0, 16)])
                pltpu.sync_copy(rows.at[0, pl.ds(0, 16)],
                                out_hbm.at[pl.ds(out_base + tb, 16)])
            plsc.subcore_barrier()

        for j in range(cc):
            k = c * cc + j
            run_chunk(k, True, flat_out, k * N_NODES)

        if with_deg:
            @pl.when(c == 0)
            def _():
                run_chunk(0, False, deg_out, 0)

    return pl.kernel(body, out_type=tuple(out_type), mesh=mesh,
                     scratch_types=scratch,
                     name=f"sc_agg{n_chunks}" + ("_deg" if with_deg else ""))


def _tc_layer(nc, agg3, h3, deg, w_cat, b_row):
    r_blk = 400
    grid = (N_NODES // r_blk,)
    n_out = D_EMB // W

    def body(a_ref, h_ref, d_ref, w_ref, b_ref, out_ref):
        rdeg = 1.0 / jnp.maximum(d_ref[:, 0:1], 1.0)
        mean = jnp.concatenate([a_ref[j] for j in range(nc)], axis=1) * rdeg
        hcat = jnp.concatenate([h_ref[j] for j in range(nc)], axis=1)
        inp = jnp.concatenate([mean, hcat], axis=1)
        o = jnp.dot(inp, w_ref[...], preferred_element_type=jnp.float32)
        o = o + b_ref[...]
        for j in range(n_out):
            out_ref[j] = o[:, j * W:(j + 1) * W]

    in_specs = [pl.BlockSpec((nc, r_blk, W), lambda i: (0, i, 0)),
                pl.BlockSpec((nc, r_blk, W), lambda i: (0, i, 0)),
                pl.BlockSpec((r_blk, W), lambda i: (i, 0)),
                pl.BlockSpec(w_cat.shape, lambda i: (0, 0)),
                pl.BlockSpec((1, D_EMB), lambda i: (0, 0))]
    return pl.pallas_call(
        body, grid=grid, in_specs=in_specs,
        out_specs=pl.BlockSpec((n_out, r_blk, W), lambda i: (0, i, 0)),
        out_shape=jax.ShapeDtypeStruct((n_out, N_NODES, W), jnp.float32),
    )(agg3, h3, deg, w_cat, b_row)


def _tc_pool_mlp(h0, h1, h2, batch3, fc1_w, fc1_b, fc2_w, fc2_b):
    r_blk = 80
    n_blk = N_NODES // r_blk
    d_cat = 3 * D_EMB
    nc = D_EMB // W

    def body(h0_r, h1_r, h2_r, b_ref, w1, b1, w2, b2, out, acc):
        i = pl.program_id(0)

        @pl.when(i == 0)
        def _():
            acc[...] = jnp.full((N_GRAPHS, d_cat), -jnp.inf, jnp.float32)

        rows = jnp.concatenate(
            [r[j] for r in (h0_r, h1_r, h2_r) for j in range(nc)], axis=1)
        gids = b_ref[0]
        g_lo = b_ref[0, 0, 0]
        g_hi = b_ref[0, r_blk - 1, 0]

        def g_body(g, carry):
            m = gids == g
            colmax = jnp.max(jnp.where(m, rows, -jnp.inf), axis=0,
                             keepdims=True)
            cur = acc[pl.ds(g, 1), :]
            acc[pl.ds(g, 1), :] = jnp.maximum(cur, colmax)
            return carry

        lax.fori_loop(g_lo, g_hi + 1, g_body, 0)

        @pl.when(i == n_blk - 1)
        def _():
            p = acc[...]
            z = jnp.maximum(
                jnp.dot(p, w1[...], preferred_element_type=jnp.float32)
                + b1[...], 0.0)
            out[...] = (jnp.dot(z, w2[...], preferred_element_type=jnp.float32)
                        + b2[...])

    h_spec = pl.BlockSpec((nc, r_blk, W), lambda i: (0, i, 0))
    in_specs = [h_spec, h_spec, h_spec,
                pl.BlockSpec((1, r_blk, 1), lambda i: (i, 0, 0)),
                pl.BlockSpec(fc1_w.shape, lambda i: (0, 0)),
                pl.BlockSpec((1, D_EMB), lambda i: (0, 0)),
                pl.BlockSpec(fc2_w.shape, lambda i: (0, 0)),
                pl.BlockSpec((1, 10), lambda i: (0, 0))]
    return pl.pallas_call(
        body, grid=(n_blk,),
        in_specs=in_specs,
        out_specs=pl.BlockSpec((N_GRAPHS, 10), lambda i: (0, 0)),
        out_shape=jax.ShapeDtypeStruct((N_GRAPHS, 10), jnp.float32),
        scratch_shapes=[pltpu.VMEM((N_GRAPHS, d_cat), jnp.float32)],
    )(h0, h1, h2, batch3, fc1_w, fc1_b, fc2_w, fc2_b)


def kernel(x, edge_index, batch, W_l0, b_l0, W_r0, W_l1, b_l1, W_r1,
           W_l2, b_l2, W_r2, fc1_W, fc1_b, fc2_W, fc2_b):
    src, dst = edge_index[0], edge_index[1]
    npad = E_PAD - N_EDGES
    pad_ar = jnp.arange(npad, dtype=jnp.int32)
    src_p = jnp.concatenate([src, (pad_ar * 37) % N_NODES])
    dst_p = jnp.concatenate([dst, N_NODES + (pad_ar % 16)])
    pk2d = ((dst_p << 16) | src_p).reshape(E_PAD // 128, 128)

    nc_in = D_IN // W
    nc_emb = D_EMB // W

    x3 = x.reshape(N_NODES, nc_in, W).transpose(1, 0, 2)
    x_flat = x3.reshape(nc_in * N_NODES, W)

    sc_in = _make_sc_agg(nc_in, with_deg=True)
    sc_emb = _make_sc_agg(nc_emb, with_deg=False)

    agg0_flat, deg = sc_in(x_flat, pk2d)
    agg0 = agg0_flat.reshape(nc_in, N_NODES, W)

    wcat0 = jnp.concatenate([W_l0, W_r0], axis=0)
    wcat1 = jnp.concatenate([W_l1, W_r1], axis=0)
    wcat2 = jnp.concatenate([W_l2, W_r2], axis=0)

    h0 = _tc_layer(nc_in, agg0, x3, deg, wcat0, b_l0.reshape(1, D_EMB))
    agg1 = sc_emb(h0.reshape(nc_emb * N_NODES, W), pk2d)[0]
    h1 = _tc_layer(nc_emb, agg1.reshape(nc_emb, N_NODES, W), h0, deg,
                   wcat1, b_l1.reshape(1, D_EMB))
    agg2 = sc_emb(h1.reshape(nc_emb * N_NODES, W), pk2d)[0]
    h2 = _tc_layer(nc_emb, agg2.reshape(nc_emb, N_NODES, W), h1, deg,
                   wcat2, b_l2.reshape(1, D_EMB))

    batch3 = batch.reshape(N_NODES // 80, 80, 1)
    return _tc_pool_mlp(h0, h1, h2, batch3,
                        fc1_W, fc1_b.reshape(1, D_EMB),
                        fc2_W, fc2_b.reshape(1, 10))
